# Initial kernel scaffold; baseline (speedup 1.0000x reference)
import jax
import jax.numpy as jnp
from jax.experimental import pallas as pl


def kernel(logits):
    raise NotImplementedError("write your pallas kernel here")


# speedup vs baseline: 2.0826x; 2.0826x over previous
import functools

import jax
import jax.numpy as jnp
from jax import lax
from jax.experimental import pallas as pl
from jax.experimental.pallas import tpu as pltpu
from jax.experimental.pallas import tpu_sc as plsc

B = 64
V = 100000
VP = 102400
NVREG = VP // 16
NGROUP = 25
TOPK = 50
CAP = 128
NEG = -3.4028235e38

_mesh = plsc.VectorSubcoreMesh(core_axis_name="c", subcore_axis_name="s")


@functools.partial(
    pl.kernel,
    mesh=_mesh,
    compiler_params=pltpu.CompilerParams(
        needs_layout_passes=False, use_tc_tiling_on_sc=False),
    out_type=(
        jax.ShapeDtypeStruct((B, 64), jnp.float32),
        jax.ShapeDtypeStruct((B, 16), jnp.int32),
    ),
    scratch_types=[
        pltpu.VMEM((VP,), jnp.float32),
        pltpu.VMEM((400,), jnp.float32),
        pltpu.VMEM((CAP,), jnp.float32),
        pltpu.VMEM((CAP,), jnp.int32),
        pltpu.VMEM((64,), jnp.float32),
        pltpu.VMEM((64,), jnp.int32),
        pltpu.VMEM((64,), jnp.float32),
        pltpu.VMEM((64,), jnp.float32),
        pltpu.VMEM((16,), jnp.int32),
        pltpu.SMEM((1,), jnp.int32),
    ],
)
def _sc_sampler(lg_hbm, g_hbm, probs_hbm, tok_hbm,
                row_v, l2_v, cand_val, cand_idx, slot_val, slot_idx,
                g_v, probs_v, tok_v, cnt_s):
    cid = lax.axis_index("c")
    sid = lax.axis_index("s")
    wid = sid * 2 + cid
    lane = lax.broadcasted_iota(jnp.int32, (16,), 0)
    negv = jnp.full((16,), NEG, jnp.float32)

    for rep in range(2):
        row = wid + rep * 32
        pltpu.sync_copy(lg_hbm.at[row * 8 + 7], row_v.at[pl.ds(0, V)])
        pltpu.sync_copy(g_hbm.at[row], g_v)

        def _pad(i, carry):
            row_v[pl.ds(V + i * 16, 16)] = negv
            return carry
        lax.fori_loop(0, (VP - V) // 16, _pad, 0)

        def _grp(h, carry):
            def _inner(j, acc):
                return jnp.maximum(acc, row_v[pl.ds((h * 256 + j) * 16, 16)])
            l2_v[pl.ds(h * 16, 16)] = lax.fori_loop(0, 256, _inner, negv)
            return carry
        lax.fori_loop(0, NGROUP, _grp, 0)

        def _round(r, tprev):
            def _mx(a, acc):
                return jnp.maximum(acc, l2_v[pl.ds(a * 16, 16)])
            mv = lax.fori_loop(0, NGROUP, _mx, negv)
            t = plsc.cummax(mv)[15]
            tv = jnp.broadcast_to(t, (16,))

            @pl.when(r < TOPK - 1)
            def _():
                def _msk(a, carry):
                    w = l2_v[pl.ds(a * 16, 16)]
                    l2_v[pl.ds(a * 16, 16)] = jnp.where(w == tv, NEG, w)
                    return carry
                lax.fori_loop(0, NGROUP, _msk, 0)
            return t
        t_c = lax.fori_loop(0, TOPK, _round, NEG)
        tcv = jnp.broadcast_to(t_c, (16,))

        for a in range(CAP // 16):
            cand_val[pl.ds(a * 16, 16)] = negv
            cand_idx[pl.ds(a * 16, 16)] = jnp.zeros((16,), jnp.int32)
        cnt_s[0] = 0

        def _comp(j, carry):
            x = row_v[pl.ds(j * 16, 16)]
            m = x >= tcv

            @pl.when(jnp.any(m))
            def _():
                c0 = cnt_s[0]
                mi = m.astype(jnp.int32)
                csum = plsc.cumsum(mi)
                pos = jnp.broadcast_to(c0, (16,)) + csum - 1
                safe = jnp.logical_and(m, pos < CAP)
                gidx = jnp.broadcast_to(j * 16, (16,)) + lane
                plsc.store_scatter(cand_idx, [pos], gidx, mask=safe)
                plsc.store_scatter(cand_val, [pos], x, mask=safe)
                cnt_s[0] = c0 + csum[15]
            return carry
        lax.fori_loop(0, NVREG, _comp, 0)

        cv = [cand_val[pl.ds(a * 16, 16)] for a in range(CAP // 16)]
        ci = [cand_idx[pl.ds(a * 16, 16)] for a in range(CAP // 16)]

        def _rank(f, ranks):
            fv = jnp.broadcast_to(f, (16,))
            vf = plsc.load_gather(cand_val, [fv])
            jf = plsc.load_gather(cand_idx, [fv])
            out = []
            for a in range(CAP // 16):
                beat = (vf > cv[a]) | ((vf == cv[a]) & (jf < ci[a]))
                out.append(ranks[a] + beat.astype(jnp.int32))
            return tuple(out)
        ranks = lax.fori_loop(
            0, CAP, _rank,
            tuple(jnp.zeros((16,), jnp.int32) for _ in range(CAP // 16)))

        for a in range(4):
            slot_val[pl.ds(a * 16, 16)] = negv
            slot_idx[pl.ds(a * 16, 16)] = jnp.zeros((16,), jnp.int32)
        for a in range(CAP // 16):
            sel = ranks[a] < 64
            plsc.store_scatter(slot_val, [ranks[a]], cv[a], mask=sel)
            plsc.store_scatter(slot_idx, [ranks[a]], ci[a], mask=sel)

        v0v = jnp.broadcast_to(slot_val[pl.ds(0, 16)][0], (16,))
        sv = [slot_val[pl.ds(a * 16, 16)] for a in range(4)]
        siv = [slot_idx[pl.ds(a * 16, 16)] for a in range(4)]
        gvv = [g_v[pl.ds(a * 16, 16)] for a in range(4)]

        psum = jnp.zeros((16,), jnp.float32)
        pvecs = []
        for a in range(4):
            glob = lane + a * 16
            p = jnp.exp(sv[a] - v0v)
            p = jnp.where(glob < TOPK, p, jnp.float32(0.0))
            pvecs.append(p)
            psum = psum + p
        Sv = jnp.broadcast_to(plsc.cumsum(psum)[15], (16,))
        for a in range(4):
            probs_v[pl.ds(a * 16, 16)] = pvecs[a] / Sv

        mvec = negv
        scs = []
        for a in range(4):
            glob = lane + a * 16
            s = (sv[a] - v0v) + gvv[a]
            s = jnp.where(glob < TOPK, s, NEG)
            scs.append(s)
            mvec = jnp.maximum(mvec, s)
        msv = jnp.broadcast_to(plsc.cummax(mvec)[15], (16,))

        selv = jnp.full((16,), 9999, jnp.int32)
        for a in range(4):
            glob = lane + a * 16
            selv = jnp.minimum(selv, jnp.where(scs[a] == msv, glob, 9999))
        sel_i = jnp.broadcast_to(-plsc.cummax(-selv)[15], (16,))

        tokv = jnp.zeros((16,), jnp.int32)
        for a in range(4):
            glob = lane + a * 16
            tokv = tokv + jnp.where(glob == sel_i, siv[a], 0)
        tok_v[pl.ds(0, 16)] = jnp.broadcast_to(plsc.cumsum(tokv)[15], (16,))

        pltpu.sync_copy(probs_v, probs_hbm.at[row])
        pltpu.sync_copy(tok_v, tok_hbm.at[row])


def kernel(logits):
    lg = logits.reshape(B * 8, V)
    g = jax.random.gumbel(jax.random.key(1234), (B, TOPK), jnp.float32)
    gp = jnp.zeros((B, 64), jnp.float32).at[:, :TOPK].set(g)
    probs_out, tok_out = _sc_sampler(lg, gp)
    return tok_out[:, 0], probs_out[:, :TOPK]


# speedup vs baseline: 3.8655x; 1.8561x over previous
import functools

import jax
import jax.numpy as jnp
from jax import lax
from jax.experimental import pallas as pl
from jax.experimental.pallas import tpu as pltpu
from jax.experimental.pallas import tpu_sc as plsc

B = 64
V = 100000
VP = 102400
NVREG = VP // 16
NL1 = 400
NGROUP = 25
TOPK = 50
CAP = 128
NEG = -3.4028235e38

_mesh = plsc.VectorSubcoreMesh(core_axis_name="c", subcore_axis_name="s")


def _treemax(vs):
    vs = list(vs)
    while len(vs) > 1:
        nxt = [jnp.maximum(vs[i], vs[i + 1]) for i in range(0, len(vs) - 1, 2)]
        if len(vs) % 2:
            nxt.append(vs[-1])
        vs = nxt
    return vs[0]


@functools.partial(
    pl.kernel,
    mesh=_mesh,
    compiler_params=pltpu.CompilerParams(
        needs_layout_passes=False, use_tc_tiling_on_sc=False),
    out_type=(
        jax.ShapeDtypeStruct((B, 64), jnp.float32),
        jax.ShapeDtypeStruct((B, 16), jnp.int32),
    ),
    scratch_types=[
        pltpu.VMEM((VP,), jnp.float32),
        pltpu.VMEM((NL1 * 16,), jnp.float32),
        pltpu.VMEM((NGROUP * 16,), jnp.float32),
        pltpu.VMEM((CAP,), jnp.float32),
        pltpu.VMEM((CAP,), jnp.int32),
        pltpu.VMEM((64,), jnp.float32),
        pltpu.VMEM((64,), jnp.int32),
        pltpu.VMEM((64,), jnp.float32),
        pltpu.VMEM((64,), jnp.float32),
        pltpu.VMEM((16,), jnp.int32),
        pltpu.SMEM((1,), jnp.int32),
    ],
)
def _sc_sampler(lg_hbm, g_hbm, probs_hbm, tok_hbm,
                row_v, l1_v, l2_v, cand_val, cand_idx, slot_val, slot_idx,
                g_v, probs_v, tok_v, cnt_s):
    cid = lax.axis_index("c")
    sid = lax.axis_index("s")
    wid = sid * 2 + cid
    lane = lax.broadcasted_iota(jnp.int32, (16,), 0)
    negv = jnp.full((16,), NEG, jnp.float32)

    def _row_body(rep, carry):
        row = wid + rep * 32
        pltpu.sync_copy(lg_hbm.at[row * 8 + 7], row_v.at[pl.ds(0, V)])
        pltpu.sync_copy(g_hbm.at[row], g_v)

        for i in range((VP - V) // 16):
            row_v[pl.ds(V + i * 16, 16)] = negv

        def _l1g(g, c):
            vs = [row_v[pl.ds((g * 16 + k) * 16, 16)] for k in range(16)]
            l1_v[pl.ds(g * 16, 16)] = _treemax(vs)
            return c
        lax.fori_loop(0, NL1, _l1g, 0)

        for h in range(NGROUP):
            vs = [l1_v[pl.ds((h * 16 + k) * 16, 16)] for k in range(16)]
            l2_v[pl.ds(h * 16, 16)] = _treemax(vs)

        def _round(r, tprev):
            vs = [l2_v[pl.ds(a * 16, 16)] for a in range(NGROUP)]
            t = plsc.cummax(_treemax(vs))[15]
            tv = jnp.broadcast_to(t, (16,))

            @pl.when(r < TOPK - 1)
            def _():
                for a in range(NGROUP):
                    l2_v[pl.ds(a * 16, 16)] = jnp.where(vs[a] == tv, NEG, vs[a])
            return t
        t_c = lax.fori_loop(0, TOPK, _round, jnp.float32(NEG))
        tcv = jnp.broadcast_to(t_c, (16,))

        for a in range(CAP // 16):
            cand_val[pl.ds(a * 16, 16)] = negv
            cand_idx[pl.ds(a * 16, 16)] = jnp.zeros((16,), jnp.int32)
        cnt_s[0] = 0

        def _comp(g, c):
            l1g = l1_v[pl.ds(g * 16, 16)]

            @pl.when(jnp.any(l1g >= tcv))
            def _():
                for k in range(16):
                    x = row_v[pl.ds((g * 16 + k) * 16, 16)]
                    m = x >= tcv

                    @pl.when(jnp.any(m))
                    def _():
                        c0 = cnt_s[0]
                        csum = plsc.cumsum(m.astype(jnp.int32))
                        pos = jnp.broadcast_to(c0, (16,)) + csum - 1
                        safe = jnp.logical_and(m, pos < CAP)
                        gidx = jnp.broadcast_to((g * 16 + k) * 16, (16,)) + lane
                        plsc.store_scatter(cand_idx, [pos], gidx, mask=safe)
                        plsc.store_scatter(cand_val, [pos], x, mask=safe)
                        cnt_s[0] = c0 + csum[15]
            return c
        lax.fori_loop(0, NL1, _comp, 0)

        cv = [cand_val[pl.ds(a * 16, 16)] for a in range(CAP // 16)]
        ci = [cand_idx[pl.ds(a * 16, 16)] for a in range(CAP // 16)]

        def _rank(f, ranks):
            fv = jnp.broadcast_to(f, (16,))
            vf = plsc.load_gather(cand_val, [fv])
            jf = plsc.load_gather(cand_idx, [fv])
            out = []
            for a in range(CAP // 16):
                beat = (vf > cv[a]) | ((vf == cv[a]) & (jf < ci[a]))
                out.append(ranks[a] + beat.astype(jnp.int32))
            return tuple(out)
        ranks = lax.fori_loop(
            0, CAP, _rank,
            tuple(jnp.zeros((16,), jnp.int32) for _ in range(CAP // 16)))

        for a in range(4):
            slot_val[pl.ds(a * 16, 16)] = negv
            slot_idx[pl.ds(a * 16, 16)] = jnp.zeros((16,), jnp.int32)
        for a in range(CAP // 16):
            sel = ranks[a] < 64
            plsc.store_scatter(slot_val, [ranks[a]], cv[a], mask=sel)
            plsc.store_scatter(slot_idx, [ranks[a]], ci[a], mask=sel)

        v0v = jnp.broadcast_to(slot_val[pl.ds(0, 16)][0], (16,))
        sv = [slot_val[pl.ds(a * 16, 16)] for a in range(4)]
        siv = [slot_idx[pl.ds(a * 16, 16)] for a in range(4)]
        gvv = [g_v[pl.ds(a * 16, 16)] for a in range(4)]

        psum = jnp.zeros((16,), jnp.float32)
        pvecs = []
        for a in range(4):
            glob = lane + a * 16
            p = jnp.exp(sv[a] - v0v)
            p = jnp.where(glob < TOPK, p, jnp.float32(0.0))
            pvecs.append(p)
            psum = psum + p
        Sv = jnp.broadcast_to(plsc.cumsum(psum)[15], (16,))
        for a in range(4):
            probs_v[pl.ds(a * 16, 16)] = pvecs[a] / Sv

        mvec = negv
        scs = []
        for a in range(4):
            glob = lane + a * 16
            s = (sv[a] - v0v) + gvv[a]
            s = jnp.where(glob < TOPK, s, NEG)
            scs.append(s)
            mvec = jnp.maximum(mvec, s)
        msv = jnp.broadcast_to(plsc.cummax(mvec)[15], (16,))

        selv = jnp.full((16,), 9999, jnp.int32)
        for a in range(4):
            glob = lane + a * 16
            selv = jnp.minimum(selv, jnp.where(scs[a] == msv, glob, 9999))
        sel_i = jnp.broadcast_to(-plsc.cummax(-selv)[15], (16,))

        tokv = jnp.zeros((16,), jnp.int32)
        for a in range(4):
            glob = lane + a * 16
            tokv = tokv + jnp.where(glob == sel_i, siv[a], 0)
        tok_v[pl.ds(0, 16)] = jnp.broadcast_to(plsc.cumsum(tokv)[15], (16,))

        pltpu.sync_copy(probs_v, probs_hbm.at[row])
        pltpu.sync_copy(tok_v, tok_hbm.at[row])
        return carry

    lax.fori_loop(0, 2, _row_body, 0)


def kernel(logits):
    lg = logits.reshape(B * 8, V)
    g = jax.random.gumbel(jax.random.key(1234), (B, TOPK), jnp.float32)
    gp = jnp.zeros((B, 64), jnp.float32).at[:, :TOPK].set(g)
    probs_out, tok_out = _sc_sampler(lg, gp)
    return tok_out[:, 0], probs_out[:, :TOPK]


# speedup vs baseline: 6.3810x; 1.6507x over previous
import functools

import jax
import jax.numpy as jnp
from jax import lax
from jax.experimental import pallas as pl
from jax.experimental.pallas import tpu as pltpu
from jax.experimental.pallas import tpu_sc as plsc

B = 64
V = 100000
VP = 102400
NVREG = VP // 16
NL1 = 400
NGROUP = 25
TOPK = 50
CAP = 128
NEG = -3.4028235e38

_mesh = plsc.VectorSubcoreMesh(core_axis_name="c", subcore_axis_name="s")


def _treemax(vs):
    vs = list(vs)
    while len(vs) > 1:
        nxt = [jnp.maximum(vs[i], vs[i + 1]) for i in range(0, len(vs) - 1, 2)]
        if len(vs) % 2:
            nxt.append(vs[-1])
        vs = nxt
    return vs[0]


@functools.partial(
    pl.kernel,
    mesh=_mesh,
    compiler_params=pltpu.CompilerParams(
        needs_layout_passes=False, use_tc_tiling_on_sc=False),
    out_type=(
        jax.ShapeDtypeStruct((B, 64), jnp.float32),
        jax.ShapeDtypeStruct((B, 16), jnp.int32),
    ),
    scratch_types=[
        pltpu.VMEM((VP,), jnp.float32),
        pltpu.VMEM((NL1 * 16,), jnp.float32),
        pltpu.VMEM((NGROUP * 16,), jnp.float32),
        pltpu.VMEM((CAP,), jnp.float32),
        pltpu.VMEM((CAP,), jnp.int32),
        pltpu.VMEM((64,), jnp.float32),
        pltpu.VMEM((64,), jnp.int32),
        pltpu.VMEM((64,), jnp.float32),
        pltpu.VMEM((64,), jnp.float32),
        pltpu.VMEM((16,), jnp.int32),
        pltpu.SMEM((1,), jnp.int32),
    ],
)
def _sc_sampler(lg_hbm, g_hbm, probs_hbm, tok_hbm,
                row_v, l1_v, l2_v, cand_val, cand_idx, slot_val, slot_idx,
                g_v, probs_v, tok_v, cnt_s):
    cid = lax.axis_index("c")
    sid = lax.axis_index("s")
    wid = sid * 2 + cid
    lane = lax.broadcasted_iota(jnp.int32, (16,), 0)
    negv = jnp.full((16,), NEG, jnp.float32)

    def _row_body(rep, carry):
        row = wid + rep * 32
        pltpu.sync_copy(lg_hbm.at[row], row_v.at[pl.ds(0, V)])
        pltpu.sync_copy(g_hbm.at[row], g_v)

        for i in range((VP - V) // 16):
            row_v[pl.ds(V + i * 16, 16)] = negv

        def _l1g(g, c):
            vs = [row_v[pl.ds((g * 16 + k) * 16, 16)] for k in range(16)]
            l1_v[pl.ds(g * 16, 16)] = _treemax(vs)
            return c
        lax.fori_loop(0, NL1, _l1g, 0)

        for h in range(NGROUP):
            vs = [l1_v[pl.ds((h * 16 + k) * 16, 16)] for k in range(16)]
            l2_v[pl.ds(h * 16, 16)] = _treemax(vs)

        def _round(r, tprev):
            vs = [l2_v[pl.ds(a * 16, 16)] for a in range(NGROUP)]
            t = plsc.cummax(_treemax(vs))[15]
            tv = jnp.broadcast_to(t, (16,))

            @pl.when(r < TOPK - 1)
            def _():
                for a in range(NGROUP):
                    l2_v[pl.ds(a * 16, 16)] = jnp.where(vs[a] == tv, NEG, vs[a])
            return t
        t_c = lax.fori_loop(0, TOPK, _round, jnp.float32(NEG))
        tcv = jnp.broadcast_to(t_c, (16,))

        for a in range(CAP // 16):
            cand_val[pl.ds(a * 16, 16)] = negv
            cand_idx[pl.ds(a * 16, 16)] = jnp.zeros((16,), jnp.int32)
        cnt_s[0] = 0

        def _comp(g, c):
            l1g = l1_v[pl.ds(g * 16, 16)]

            @pl.when(jnp.any(l1g >= tcv))
            def _():
                for k in range(16):
                    x = row_v[pl.ds((g * 16 + k) * 16, 16)]
                    m = x >= tcv

                    @pl.when(jnp.any(m))
                    def _():
                        c0 = cnt_s[0]
                        csum = plsc.cumsum(m.astype(jnp.int32))
                        pos = jnp.broadcast_to(c0, (16,)) + csum - 1
                        safe = jnp.logical_and(m, pos < CAP)
                        gidx = jnp.broadcast_to((g * 16 + k) * 16, (16,)) + lane
                        plsc.store_scatter(cand_idx, [pos], gidx, mask=safe)
                        plsc.store_scatter(cand_val, [pos], x, mask=safe)
                        cnt_s[0] = c0 + csum[15]
            return c
        lax.fori_loop(0, NL1, _comp, 0)

        cv = [cand_val[pl.ds(a * 16, 16)] for a in range(CAP // 16)]
        ci = [cand_idx[pl.ds(a * 16, 16)] for a in range(CAP // 16)]

        def _rank(f, ranks):
            fv = jnp.broadcast_to(f, (16,))
            vf = plsc.load_gather(cand_val, [fv])
            jf = plsc.load_gather(cand_idx, [fv])
            out = []
            for a in range(CAP // 16):
                beat = (vf > cv[a]) | ((vf == cv[a]) & (jf < ci[a]))
                out.append(ranks[a] + beat.astype(jnp.int32))
            return tuple(out)
        ranks = lax.fori_loop(
            0, CAP, _rank,
            tuple(jnp.zeros((16,), jnp.int32) for _ in range(CAP // 16)))

        for a in range(4):
            slot_val[pl.ds(a * 16, 16)] = negv
            slot_idx[pl.ds(a * 16, 16)] = jnp.zeros((16,), jnp.int32)
        for a in range(CAP // 16):
            sel = ranks[a] < 64
            plsc.store_scatter(slot_val, [ranks[a]], cv[a], mask=sel)
            plsc.store_scatter(slot_idx, [ranks[a]], ci[a], mask=sel)

        v0v = jnp.broadcast_to(slot_val[pl.ds(0, 16)][0], (16,))
        sv = [slot_val[pl.ds(a * 16, 16)] for a in range(4)]
        siv = [slot_idx[pl.ds(a * 16, 16)] for a in range(4)]
        gvv = [g_v[pl.ds(a * 16, 16)] for a in range(4)]

        psum = jnp.zeros((16,), jnp.float32)
        pvecs = []
        for a in range(4):
            glob = lane + a * 16
            p = jnp.exp(sv[a] - v0v)
            p = jnp.where(glob < TOPK, p, jnp.float32(0.0))
            pvecs.append(p)
            psum = psum + p
        Sv = jnp.broadcast_to(plsc.cumsum(psum)[15], (16,))
        for a in range(4):
            probs_v[pl.ds(a * 16, 16)] = pvecs[a] / Sv

        mvec = negv
        scs = []
        for a in range(4):
            glob = lane + a * 16
            s = (sv[a] - v0v) + gvv[a]
            s = jnp.where(glob < TOPK, s, NEG)
            scs.append(s)
            mvec = jnp.maximum(mvec, s)
        msv = jnp.broadcast_to(plsc.cummax(mvec)[15], (16,))

        selv = jnp.full((16,), 9999, jnp.int32)
        for a in range(4):
            glob = lane + a * 16
            selv = jnp.minimum(selv, jnp.where(scs[a] == msv, glob, 9999))
        sel_i = jnp.broadcast_to(-plsc.cummax(-selv)[15], (16,))

        tokv = jnp.zeros((16,), jnp.int32)
        for a in range(4):
            glob = lane + a * 16
            tokv = tokv + jnp.where(glob == sel_i, siv[a], 0)
        tok_v[pl.ds(0, 16)] = jnp.broadcast_to(plsc.cumsum(tokv)[15], (16,))

        pltpu.sync_copy(probs_v, probs_hbm.at[row])
        pltpu.sync_copy(tok_v, tok_hbm.at[row])
        return carry

    lax.fori_loop(0, 2, _row_body, 0)


def kernel(logits):
    lg = logits[:, -1]
    g = jax.random.gumbel(jax.random.key(1234), (B, TOPK), jnp.float32)
    gp = jnp.zeros((B, 64), jnp.float32).at[:, :TOPK].set(g)
    probs_out, tok_out = _sc_sampler(lg, gp)
    return tok_out[:, 0], probs_out[:, :TOPK]
